# Initial kernel scaffold; baseline (speedup 1.0000x reference)
import jax
import jax.numpy as jnp
from jax.experimental import pallas as pl


def kernel(input_batch, table):
    raise NotImplementedError("write your pallas kernel here")


# speedup vs baseline: 6.2421x; 6.2421x over previous
import jax
import jax.numpy as jnp
from jax import lax
from jax.experimental import pallas as pl
from jax.experimental.pallas import tpu as pltpu
from jax.experimental.pallas import tpu_sc as plsc

NC = 2
NS = 16
NW = NC * NS
CH = 128
NBUF = 4


def _body(idx_hbm, table_hbm, out_hbm, idx_v, rows_v, gsem, osem):
    cpw = idx_v.shape[0]
    wid = lax.axis_index("s") * NC + lax.axis_index("c")
    row_base = wid * cpw * CH

    pltpu.sync_copy(idx_hbm.at[wid], idx_v)

    def g_desc(chunk, slot):
        return pltpu.make_async_copy(
            table_hbm.at[idx_v.at[chunk]], rows_v.at[slot], gsem.at[slot])

    def s_desc(chunk, slot):
        return pltpu.make_async_copy(
            rows_v.at[slot], out_hbm.at[pl.ds(row_base + chunk * CH, CH)],
            osem.at[slot])

    for b in range(NBUF):
        g_desc(b, b).start()

    def outer(it, carry):
        jo = it * NBUF
        for b in range(NBUF):
            j = jo + b
            g_desc(j, b).wait()
            s_desc(j, b).start()
            jn = j + NBUF

            @pl.when(jn < cpw)
            def _():
                s_desc(j, b).wait()
                g_desc(jn, b).start()
        return carry

    lax.fori_loop(0, cpw // NBUF, outer, 0)

    for b in range(NBUF):
        s_desc(cpw - NBUF + b, b).wait()


def kernel(input_batch, table):
    batch, hist = input_batch.shape
    _, dim = table.shape
    total = batch * hist
    cpw = total // (NW * CH)
    idx = input_batch.reshape(NW, cpw, CH)
    mesh = plsc.VectorSubcoreMesh(
        core_axis_name="c", subcore_axis_name="s",
        num_cores=NC, num_subcores=NS)
    out = pl.kernel(
        _body,
        out_type=jax.ShapeDtypeStruct((total, dim), jnp.float32),
        mesh=mesh,
        compiler_params=pltpu.CompilerParams(use_tc_tiling_on_sc=False),
        scratch_types=[
            pltpu.VMEM((cpw, CH), jnp.int32),
            pltpu.VMEM((NBUF, CH, dim), jnp.float32),
            pltpu.SemaphoreType.DMA((NBUF,)),
            pltpu.SemaphoreType.DMA((NBUF,)),
        ],
    )(idx, table)
    return out.reshape(batch, hist, dim)
